# Initial kernel scaffold; baseline (speedup 1.0000x reference)
import functools

import jax
import jax.numpy as jnp
from jax import lax
from jax.experimental import pallas as pl
from jax.experimental.pallas import tpu as pltpu
from jax.experimental.pallas import tpu_sc as plsc

_NUM_EMB = 1000000
_DIM = 32
_BATCH = 16384
_HIST = 50
_TOTAL = _BATCH * _HIST

_NC = 2
_NS = 16
_NW = _NC * _NS
_BPW = _TOTAL // _NW

_GATHER = 128
_CHUNK = 1024
_SUB = _CHUNK // _GATHER
_NCHUNK = _BPW // _CHUNK
_IDX_ROWS = _BPW // _GATHER

_mesh = plsc.VectorSubcoreMesh(core_axis_name="c", subcore_axis_name="s")


@functools.partial(
    pl.kernel,
    mesh=_mesh,
    out_type=jax.ShapeDtypeStruct((_TOTAL, _DIM), jnp.float32),
    scratch_types=[
        pltpu.VMEM((_IDX_ROWS, _GATHER), jnp.int32),
        pltpu.VMEM((2, _CHUNK, _DIM), jnp.float32),
        pltpu.SemaphoreType.DMA,
        pltpu.SemaphoreType.DMA,
        pltpu.SemaphoreType.DMA,
    ],
)
def _embed_sc(idx_hbm, table_hbm, out_hbm, idx_v, rows_v, gsem0, gsem1, osem):
    wid = lax.axis_index("s") * _NC + lax.axis_index("c")
    row0 = wid * _IDX_ROWS
    base = wid * _BPW

    pltpu.sync_copy(idx_hbm.at[pl.ds(row0, _IDX_ROWS)], idx_v)

    def chunk(i, carry):
        copies = []
        for j in range(_SUB):
            copies.append(
                pltpu.async_copy(
                    table_hbm.at[idx_v.at[i * _SUB + j]],
                    rows_v.at[0, pl.ds(j * _GATHER, _GATHER)],
                    gsem0,
                )
            )
        for c in copies:
            c.wait()
        pltpu.sync_copy(rows_v.at[0], out_hbm.at[pl.ds(base + i * _CHUNK, _CHUNK)])
        return carry

    lax.fori_loop(0, _NCHUNK, chunk, 0)


def kernel(input_, weight):
    idx = input_.astype(jnp.int32).reshape(_TOTAL // _GATHER, _GATHER)
    out = _embed_sc(idx, weight)
    return out.reshape(_BATCH, _HIST, _DIM)


# speedup vs baseline: 1.1024x; 1.1024x over previous
import functools

import jax
import jax.numpy as jnp
from jax import lax
from jax.experimental import pallas as pl
from jax.experimental.pallas import tpu as pltpu
from jax.experimental.pallas import tpu_sc as plsc

_NUM_EMB = 1000000
_DIM = 32
_BATCH = 16384
_HIST = 50
_TOTAL = _BATCH * _HIST

_NC = 2
_NS = 16
_NW = _NC * _NS
_BPW = _TOTAL // _NW

_GATHER = 128
_CHUNK = 1024
_SUB = _CHUNK // _GATHER
_NCHUNK = _BPW // _CHUNK
_IDX_ROWS = _BPW // _GATHER

_mesh = plsc.VectorSubcoreMesh(core_axis_name="c", subcore_axis_name="s")


@functools.partial(
    pl.kernel,
    mesh=_mesh,
    out_type=jax.ShapeDtypeStruct((_TOTAL, _DIM), jnp.float32),
    compiler_params=pltpu.CompilerParams(use_tc_tiling_on_sc=False),
    scratch_types=[
        pltpu.VMEM((_IDX_ROWS, _GATHER), jnp.int32),
        pltpu.VMEM((2, _CHUNK, _DIM), jnp.float32),
        pltpu.SemaphoreType.DMA,
        pltpu.SemaphoreType.DMA,
        pltpu.SemaphoreType.DMA,
    ],
)
def _embed_sc(idx_hbm, table_hbm, out_hbm, idx_v, rows_v, gsem0, gsem1, osem):
    wid = lax.axis_index("s") * _NC + lax.axis_index("c")
    row0 = wid * _IDX_ROWS
    base = wid * _BPW

    pltpu.sync_copy(idx_hbm.at[pl.ds(row0, _IDX_ROWS)], idx_v)

    def chunk(i, carry):
        copies = []
        for j in range(_SUB):
            copies.append(
                pltpu.async_copy(
                    table_hbm.at[idx_v.at[i * _SUB + j]],
                    rows_v.at[0, pl.ds(j * _GATHER, _GATHER)],
                    gsem0,
                )
            )
        for c in copies:
            c.wait()
        pltpu.sync_copy(rows_v.at[0], out_hbm.at[pl.ds(base + i * _CHUNK, _CHUNK)])
        return carry

    lax.fori_loop(0, _NCHUNK, chunk, 0)


def kernel(input_, weight):
    idx = input_.astype(jnp.int32).reshape(_TOTAL // _GATHER, _GATHER)
    out = _embed_sc(idx, weight)
    return out.reshape(_BATCH, _HIST, _DIM)


# speedup vs baseline: 1.1137x; 1.0102x over previous
import functools

import jax
import jax.numpy as jnp
from jax import lax
from jax.experimental import pallas as pl
from jax.experimental.pallas import tpu as pltpu
from jax.experimental.pallas import tpu_sc as plsc

_NUM_EMB = 1000000
_DIM = 32
_BATCH = 16384
_HIST = 50
_TOTAL = _BATCH * _HIST

_NC = 2
_NS = 16
_NW = _NC * _NS
_BPW = _TOTAL // _NW

_GATHER = 128
_CHUNK = 640
_SUB = _CHUNK // _GATHER
_NCHUNK = _BPW // _CHUNK
_IDX_ROWS = _BPW // _GATHER
_NBUF = 4

_mesh = plsc.VectorSubcoreMesh(core_axis_name="c", subcore_axis_name="s")


@functools.partial(
    pl.kernel,
    mesh=_mesh,
    out_type=jax.ShapeDtypeStruct((_TOTAL, _DIM), jnp.float32),
    compiler_params=pltpu.CompilerParams(use_tc_tiling_on_sc=False),
    scratch_types=[
        pltpu.VMEM((_IDX_ROWS, _GATHER), jnp.int32),
        pltpu.VMEM((_NBUF, _CHUNK, _DIM), jnp.float32),
        [pltpu.SemaphoreType.DMA] * _NBUF,
        [pltpu.SemaphoreType.DMA] * _NBUF,
    ],
)
def _embed_sc(idx_hbm, table_hbm, out_hbm, idx_v, rows_v, gsems, osems):
    wid = lax.axis_index("s") * _NC + lax.axis_index("c")
    base = wid * _BPW
    row0 = wid * _IDX_ROWS

    pltpu.sync_copy(idx_hbm.at[pl.ds(row0, _IDX_ROWS)], idx_v)

    def fire(ci, b):
        for j in range(_SUB):
            pltpu.async_copy(
                table_hbm.at[idx_v.at[ci * _SUB + j]],
                rows_v.at[b, pl.ds(j * _GATHER, _GATHER)],
                gsems[b],
            )

    def drain_g(b):
        pltpu.make_async_copy(out_hbm.at[pl.ds(0, _CHUNK)], rows_v.at[b], gsems[b]).wait()

    def fire_out(ci, b):
        pltpu.async_copy(rows_v.at[b], out_hbm.at[pl.ds(base + ci * _CHUNK, _CHUNK)], osems[b])

    def drain_o(b):
        pltpu.make_async_copy(rows_v.at[b], out_hbm.at[pl.ds(0, _CHUNK)], osems[b]).wait()

    fire(0, 0)
    fire(1, 1)
    fire(2, 2)
    drain_g(0); fire_out(0, 0); fire(3, 3)
    drain_g(1); fire_out(1, 1); drain_o(0); fire(4, 0)
    drain_g(2); fire_out(2, 2); drain_o(1); fire(5, 1)

    def group(k, carry):
        for t in range(4):
            b = (3 + t) % 4
            ci = 3 + 4 * k + t
            drain_g(b)
            fire_out(ci, b)
            b3 = (b + 3) % 4
            drain_o(b3)
            fire(ci + 3, b3)
        return carry

    lax.fori_loop(0, (_NCHUNK - 8) // 4, group, 0)

    drain_g(3); fire_out(35, 3); drain_o(2); fire(38, 2)
    drain_g(0); fire_out(36, 0); drain_o(3); fire(39, 3)
    drain_g(1); fire_out(37, 1)
    drain_g(2); fire_out(38, 2)
    drain_g(3); fire_out(39, 3)
    drain_o(0); drain_o(1); drain_o(2); drain_o(3)


def kernel(input_, weight):
    idx = input_.astype(jnp.int32).reshape(_TOTAL // _GATHER, _GATHER)
    out = _embed_sc(idx, weight)
    return out.reshape(_BATCH, _HIST, _DIM)


# speedup vs baseline: 1.6485x; 1.4803x over previous
import functools

import jax
import jax.numpy as jnp
from jax import lax
from jax.experimental import pallas as pl
from jax.experimental.pallas import tpu as pltpu
from jax.experimental.pallas import tpu_sc as plsc

_NUM_EMB = 1000000
_DIM = 32
_BATCH = 16384
_HIST = 50

_NC = 2
_NS = 16
_NW = _NC * _NS
_BPW = _BATCH // _NW
_G = 128
_NG = _BPW // _G
_BT = _BATCH // _G
_DT = _DIM // 8

_mesh = plsc.VectorSubcoreMesh(core_axis_name="c", subcore_axis_name="s")


@functools.partial(
    pl.kernel,
    mesh=_mesh,
    out_type=jax.ShapeDtypeStruct((_HIST, _DT, _BT, 8, _G), jnp.float32),
    compiler_params=pltpu.CompilerParams(
        use_tc_tiling_on_sc=False, needs_layout_passes=False
    ),
    scratch_types=[
        pltpu.VMEM((_HIST, _NG, _G), jnp.int32),
        pltpu.VMEM((2, _BPW, _DIM), jnp.float32),
        pltpu.VMEM((2, _DT, _NG, 8, _G), jnp.float32),
        [pltpu.SemaphoreType.DMA] * 2,
        [pltpu.SemaphoreType.DMA] * 2,
    ],
)
def _embed_sc(idx_hbm, table_hbm, out_hbm, idx_v, rows_v, trows_v, gsems, osems):
    wid = lax.axis_index("s") * _NC + lax.axis_index("c")

    pltpu.sync_copy(idx_hbm.at[:, pl.ds(wid * _NG, _NG)], idx_v)

    lanes = lax.iota(jnp.int32, 16)

    def fire(h, b):
        for k in range(_NG):
            pltpu.async_copy(
                table_hbm.at[idx_v.at[h, k]],
                rows_v.at[b, pl.ds(k * _G, _G)],
                gsems[b],
            )

    def drain_g(b):
        pltpu.make_async_copy(table_hbm.at[pl.ds(0, _BPW)], rows_v.at[b], gsems[b]).wait()

    def transpose(b):
        def step(i, carry):
            dt = i // 8
            ds_ = i % 8
            for bt in range(_NG):
                for g in range(_G // 16):
                    rows = plsc.load_gather(
                        rows_v.at[b],
                        [bt * _G + g * 16 + lanes,
                         jnp.full((16,), dt * 8 + ds_, jnp.int32)],
                    )
                    trows_v[b, dt, bt, ds_, pl.ds(g * 16, 16)] = rows
            return carry

        lax.fori_loop(0, 32, step, 0)

    def fire_out(h, b):
        pltpu.async_copy(
            trows_v.at[b],
            out_hbm.at[h, :, pl.ds(wid * _NG, _NG)],
            osems[b],
        )

    def drain_o(b):
        pltpu.make_async_copy(trows_v.at[b], out_hbm.at[0, :, pl.ds(0, _NG)], osems[b]).wait()

    fire(0, 0)
    drain_g(0); fire(1, 1); transpose(0); fire_out(0, 0)
    drain_g(1); fire(2, 0); transpose(1); fire_out(1, 1)

    def pair(i, carry):
        h = 2 * i
        drain_g(0); fire(h + 1, 1); drain_o(0); transpose(0); fire_out(h, 0)
        drain_g(1); fire(h + 2, 0); drain_o(1); transpose(1); fire_out(h + 1, 1)
        return carry

    lax.fori_loop(1, 24, pair, 0)

    drain_g(0); fire(49, 1); drain_o(0); transpose(0); fire_out(48, 0)
    drain_g(1); drain_o(1); transpose(1); fire_out(49, 1)
    drain_o(0); drain_o(1)


def kernel(input_, weight):
    idx = input_.astype(jnp.int32).T.reshape(_HIST, _BT, _G)
    out5 = _embed_sc(idx, weight)
    return out5.transpose(2, 4, 0, 1, 3).reshape(_BATCH, _HIST, _DIM)


# speedup vs baseline: 2.2253x; 1.3499x over previous
import functools

import jax
import jax.numpy as jnp
from jax import lax
from jax.experimental import pallas as pl
from jax.experimental.pallas import tpu as pltpu
from jax.experimental.pallas import tpu_sc as plsc

_NUM_EMB = 1000000
_DIM = 32
_BATCH = 16384
_HIST = 50

_NC = 2
_NS = 16
_NW = _NC * _NS
_BPW = _BATCH // _NW
_G = 128
_NG = _BPW // _G
_BT = _BATCH // _G
_DT = _DIM // 8

_mesh = plsc.VectorSubcoreMesh(core_axis_name="c", subcore_axis_name="s")


@functools.partial(
    pl.kernel,
    mesh=_mesh,
    out_type=jax.ShapeDtypeStruct((_HIST, _DT, _BT, 8, _G), jnp.float32),
    compiler_params=pltpu.CompilerParams(
        use_tc_tiling_on_sc=False, needs_layout_passes=False
    ),
    scratch_types=[
        pltpu.VMEM((_HIST, _NG, _G), jnp.int32),
        pltpu.VMEM((2, _BPW, _DIM), jnp.float32),
        pltpu.VMEM((2, _DT, _NG, 8, _G), jnp.float32),
        [pltpu.SemaphoreType.DMA] * 2,
        [pltpu.SemaphoreType.DMA] * 2,
    ],
)
def _embed_sc(idx_hbm, table_hbm, out_hbm, idx_v, rows_v, trows_v, gsems, osems):
    wid = lax.axis_index("s") * _NC + lax.axis_index("c")

    pltpu.sync_copy(idx_hbm.at[:, pl.ds(wid * _NG, _NG)], idx_v)

    lanes = lax.iota(jnp.int32, 16)

    def fire(h, b):
        for k in range(_NG):
            pltpu.async_copy(
                table_hbm.at[idx_v.at[h, k]],
                rows_v.at[b, pl.ds(k * _G, _G)],
                gsems[b],
            )

    def drain_g(b):
        pltpu.make_async_copy(table_hbm.at[pl.ds(0, _BPW)], rows_v.at[b], gsems[b]).wait()

    def transpose(b):
        @plsc.parallel_loop(0, _DIM, unroll=4)
        def step(d):
            for bt in range(_NG):
                for g in range(_G // 16):
                    rows = plsc.load_gather(
                        rows_v.at[b],
                        [bt * _G + g * 16 + lanes,
                         jnp.full((16,), d, jnp.int32)],
                    )
                    trows_v[b, d // 8, bt, d % 8, pl.ds(g * 16, 16)] = rows

    def fire_out(h, b):
        pltpu.async_copy(
            trows_v.at[b],
            out_hbm.at[h, :, pl.ds(wid * _NG, _NG)],
            osems[b],
        )

    def drain_o(b):
        pltpu.make_async_copy(trows_v.at[b], out_hbm.at[0, :, pl.ds(0, _NG)], osems[b]).wait()

    fire(0, 0)

    def pair(i, carry):
        h = 2 * i
        drain_g(0)
        fire(h + 1, 1)

        @pl.when(i > 0)
        def _():
            drain_o(0)

        transpose(0)
        fire_out(h, 0)

        drain_g(1)

        @pl.when(i < _HIST // 2 - 1)
        def _():
            fire(h + 2, 0)

        @pl.when(i > 0)
        def _():
            drain_o(1)

        transpose(1)
        fire_out(h + 1, 1)
        return carry

    lax.fori_loop(0, _HIST // 2, pair, 0)
    drain_o(0)
    drain_o(1)


def kernel(input_, weight):
    idx = input_.astype(jnp.int32).T.reshape(_HIST, _BT, _G)
    out5 = _embed_sc(idx, weight)
    return out5.transpose(2, 4, 0, 1, 3).reshape(_BATCH, _HIST, _DIM)
